# baseline (speedup 1.0000x reference)
import functools

import jax
import jax.numpy as jnp
from jax import lax
from jax.experimental import pallas as pl
from jax.experimental.pallas import tpu as pltpu
from jax.experimental.pallas import tpu_sc as plsc

BATCH = 16384
HIST = 50
D = 64

NC, NS = 2, 16
NW = NC * NS
BW = BATCH // NW
NBG = BW // 128
CB = 2
NJ = NBG // CB
G = HIST * NJ
NBUF = 5
K = 4


def _emb_body(ids_hbm, table_hbm, out_hbm, ids_v, rows_v, *sems):
    gat_sems = sems[: CB * NBUF]
    st_sems = sems[CB * NBUF :]
    wid = lax.axis_index("s") * NC + lax.axis_index("c")

    pltpu.sync_copy(ids_hbm.at[:, pl.ds(wid * NBG, NBG), :], ids_v)

    def issue_gather(g, s):
        h = g // NJ
        jj = g % NJ
        for e in range(CB):
            pltpu.async_copy(
                table_hbm.at[ids_v.at[h, jj * CB + e]],
                rows_v.at[s, pl.ds(e * 128, 128), :],
                gat_sems[CB * s + e],
            )

    def wait_gather(s):
        for e in range(CB):
            pltpu.make_async_copy(
                table_hbm.at[pl.ds(0, 128)],
                rows_v.at[s, pl.ds(e * 128, 128), :],
                gat_sems[CB * s + e],
            ).wait()

    def issue_store(g, s):
        h = g // NJ
        jj = g % NJ
        bb = wid * NBG + jj * CB
        pltpu.async_copy(
            rows_v.at[s],
            out_hbm.at[pl.ds(bb * 128, CB * 128), h, :],
            st_sems[s],
        )

    def wait_store(s):
        pltpu.make_async_copy(
            rows_v.at[s], out_hbm.at[pl.ds(0, CB * 128), 0, :], st_sems[s]
        ).wait()

    for s in range(K):
        issue_gather(s, s)

    def outer(gg, carry):
        for i in range(NBUF):
            g = gg * NBUF + i
            gp = g + K
            s2 = (i + K) % NBUF

            @pl.when(jnp.logical_and(gp < G, gp >= NBUF))
            def _():
                wait_store(s2)

            @pl.when(gp < G)
            def _():
                issue_gather(gp, s2)

            wait_gather(i)
            issue_store(g, i)
        return carry

    lax.fori_loop(0, G // NBUF, outer, 0)

    for s in range(NBUF):
        wait_store(s)


_mesh = plsc.VectorSubcoreMesh(
    core_axis_name="c", subcore_axis_name="s", num_cores=NC, num_subcores=NS
)

_emb = functools.partial(
    pl.kernel,
    out_type=jax.ShapeDtypeStruct((BATCH, HIST, D), jnp.float32),
    mesh=_mesh,
    scratch_types=[
        pltpu.VMEM((HIST, NBG, 128), jnp.int32),
        pltpu.VMEM((NBUF, CB * 128, D), jnp.float32),
    ]
    + [pltpu.SemaphoreType.DMA] * ((CB + 1) * NBUF),
    compiler_params=pltpu.CompilerParams(use_tc_tiling_on_sc=False),
)(_emb_body)


def kernel(entity_ids, table):
    ids3 = entity_ids.astype(jnp.int32).T.reshape(HIST, BATCH // 128, 128)
    return _emb(ids3, table)
